# baseline (speedup 1.0000x reference)
import functools

import jax
import jax.numpy as jnp
from jax import lax
from jax.experimental import pallas as pl
from jax.experimental.pallas import tpu as pltpu
from jax.experimental.pallas import tpu_sc as plsc

N = 50000
E = 600000
D = 128
HID = 16
HEADS = 8
OUT = 128

NC = 2
NS = 16
NRANGE = 16
RNG = 3200
NOUT = NRANGE * RNG
RPACK = RNG // 16
ACCROWS = RNG + RPACK
EPW = 37504
EPAD = NS * EPW
NCHUNK = 8
CH = EPW // NCHUNK
SELCAP = 3072
G = 64
SHARE = RNG // NS


def _mm_kernel(x_ref, w_ref, b_ref, o_ref):
    o_ref[...] = jnp.dot(x_ref[...], w_ref[...],
                         preferred_element_type=jnp.float32) + b_ref[...]


def _matmul_bias(x, w, b):
    n, d = x.shape
    dout = w.shape[1]
    blk = 1000
    return pl.pallas_call(
        _mm_kernel,
        grid=(n // blk,),
        in_specs=[
            pl.BlockSpec((blk, d), lambda i: (i, 0)),
            pl.BlockSpec((d, dout), lambda i: (0, 0)),
            pl.BlockSpec((1, dout), lambda i: (0, 0)),
        ],
        out_specs=pl.BlockSpec((blk, dout), lambda i: (i, 0)),
        out_shape=jax.ShapeDtypeStruct((n, dout), jnp.float32),
    )(x, w, b.reshape(1, dout))


def _fin_kernel(s_ref, m_ref, b_ref, o_ref):
    s = s_ref[...]
    m = m_ref[...]
    parts = []
    for h in range(8):
        parts.append(m[:, h * 16:(h + 1) * 16] / (s[:, h:h + 1] + 1e-16))
    x = jnp.concatenate(parts, axis=1) + b_ref[...]
    o_ref[...] = jnp.where(x > 0, x, jnp.exp(jnp.minimum(x, 0.0)) - 1.0)


def _finalize(s, m, b):
    blk = 1000
    return pl.pallas_call(
        _fin_kernel,
        grid=(N // blk,),
        in_specs=[
            pl.BlockSpec((blk, 8), lambda i: (i, 0)),
            pl.BlockSpec((blk, 128), lambda i: (i, 0)),
            pl.BlockSpec((1, 128), lambda i: (0, 0)),
        ],
        out_specs=pl.BlockSpec((blk, 128), lambda i: (i, 0)),
        out_shape=jax.ShapeDtypeStruct((N, 128), jnp.float32),
    )(s, m, b.reshape(1, 128))


def _make_sc_pass(heads, ch):
    mesh = plsc.VectorSubcoreMesh(core_axis_name="c", subcore_axis_name="s")

    @functools.partial(
        pl.kernel,
        out_type=(jax.ShapeDtypeStruct((NRANGE * RPACK, 128), jnp.float32),
                  jax.ShapeDtypeStruct((NOUT, 128), jnp.float32)),
        mesh=mesh,
        compiler_params=pltpu.CompilerParams(needs_layout_passes=False),
        scratch_types=[
            pltpu.VMEM_SHARED((ACCROWS, 128), jnp.float32),
            pltpu.VMEM((CH,), jnp.int32),
            pltpu.VMEM((CH,), jnp.int32),
            pltpu.VMEM((SELCAP,), jnp.int32),
            pltpu.VMEM((SELCAP,), jnp.int32),
            pltpu.VMEM((G,), jnp.int32),
            pltpu.VMEM((G,), jnp.int32),
            pltpu.VMEM((G,), jnp.int32),
            pltpu.VMEM((G,), jnp.int32),
            pltpu.VMEM((G, 128), jnp.float32),
            pltpu.VMEM((G, 128), jnp.float32),
            pltpu.VMEM((G, 128), jnp.float32),
            pltpu.VMEM((G, 128), jnp.float32),
            pltpu.VMEM((2 * G, 128), jnp.float32),
            pltpu.VMEM((2 * G,), jnp.int32),
            pltpu.VMEM((128,), jnp.float32),
            pltpu.SemaphoreType.DMA,
            pltpu.SemaphoreType.DMA,
            pltpu.SemaphoreType.DMA,
            pltpu.SemaphoreType.DMA,
        ],
    )
    def sc_pass(xl_hbm, xr_hbm, src_hbm, dst_hbm, att_hbm, zm_hbm,
                outP_hbm, outM_hbm,
                accM, sbuf_s, sbuf_d, sels, seld,
                gidx_s0, gidx_d0, gidx_s1, gidx_d1,
                xlbuf0, xrbuf0, xlbuf1, xrbuf1,
                vbufC, idxc, attv,
                sem_l0, sem_r0, sem_l1, sem_r1):
        cid = lax.axis_index("c")
        wid = lax.axis_index("s")
        iota = lax.iota(jnp.int32, 16)

        pltpu.sync_copy(att_hbm, attv)
        pltpu.sync_copy(zm_hbm.at[pl.ds(0, G)], vbufC.at[pl.ds(G, G)])

        gbufs = ((gidx_s0, gidx_d0, xlbuf0, xrbuf0, sem_l0, sem_r0),
                 (gidx_s1, gidx_d1, xlbuf1, xrbuf1, sem_l1, sem_r1))

        def issue_gathers(g, p):
            gs, gd, xb, rb, sl, sr = gbufs[p]

            def idx_copy(t, _):
                gs[pl.ds(t * 16, 16)] = sels[pl.ds(g * G + t * 16, 16)]
                gd[pl.ds(t * 16, 16)] = seld[pl.ds(g * G + t * 16, 16)]
                return 0

            lax.fori_loop(0, G // 16, idx_copy, 0)
            pltpu.async_copy(xl_hbm.at[gs], xb, sl)
            pltpu.async_copy(xr_hbm.at[gd], rb, sr)

        def wait_gathers(p):
            gs, gd, xb, rb, sl, sr = gbufs[p]
            pltpu.make_async_copy(xl_hbm.at[gs], xb, sl).wait()
            pltpu.make_async_copy(xr_hbm.at[gd], rb, sr).wait()

        def range_body(r, _):
            base = (cid * (NRANGE // 2) + r) * RNG
            rid = cid * (NRANGE // 2) + r
            pltpu.sync_copy(zm_hbm.at[pl.ds(0, SHARE)],
                            accM.at[pl.ds(wid * SHARE, SHARE)])

            @pl.when(wid == 0)
            def _():
                pltpu.sync_copy(zm_hbm.at[pl.ds(0, RPACK)],
                                accM.at[pl.ds(RNG, RPACK)])

            plsc.subcore_barrier()

            def chunk_body(k, cursor):
                off = wid * EPW + k * CH
                pltpu.sync_copy(src_hbm.at[pl.ds(off, CH)], sbuf_s)
                pltpu.sync_copy(dst_hbm.at[pl.ds(off, CH)], sbuf_d)

                def vec_body(i, cur):
                    dv = sbuf_d[pl.ds(i * 16, 16)]
                    sv = sbuf_s[pl.ds(i * 16, 16)]
                    m = (dv >= base) & (dv < base + RNG)
                    mi = m.astype(jnp.int32)
                    cum = plsc.cumsum(mi)
                    pos = cur + cum - 1
                    pos = jnp.where(m, pos, SELCAP - 16 + iota)
                    plsc.store_scatter(seld, [pos], dv, mask=m)
                    plsc.store_scatter(sels, [pos], sv, mask=m)
                    return jnp.minimum(cur + cum[15], SELCAP - G - 16)

                return lax.fori_loop(0, CH // 16, vec_body, cursor)

            cnt = lax.fori_loop(0, NCHUNK, chunk_body, 0)

            padd = jnp.full((16,), base, jnp.int32)
            pads = jnp.zeros((16,), jnp.int32)

            def pad_body(k, _):
                seld[pl.ds(cnt + k * 16, 16)] = padd
                sels[pl.ds(cnt + k * 16, 16)] = pads
                return 0

            lax.fori_loop(0, G // 16, pad_body, 0)

            nsup = (cnt + G - 1) // G

            def compute_and_scatter(g, p):
                _, _, xb, rb, _, _ = gbufs[p]

                def sub_body(j, _):
                    rows = j * 16 + iota
                    dv = seld[pl.ds(g * G + j * 16, 16)]
                    dstl = dv - base
                    idxc[pl.ds(j * 16, 16)] = dstl
                    idxc[pl.ds(G + j * 16, 16)] = RNG + (dstl >> 4)
                    colbase = (dstl & 15) * 8
                    valid = (g * G + j * 16 + iota) < cnt
                    neg = jnp.where(valid, 0.0, -1e30)
                    for h in range(heads):
                        def ch_body(t, eacc, h=h):
                            c = h * ch + t
                            cv = jnp.full((16,), c, jnp.int32)
                            av = plsc.load_gather(xb, [rows, cv])
                            bv = plsc.load_gather(rb, [rows, cv])
                            atv = plsc.load_gather(attv, [cv])
                            xe = av + bv
                            lr = jnp.maximum(xe, xe * 0.2)
                            return eacc + lr * atv

                        e = lax.fori_loop(0, ch, ch_body,
                                          jnp.zeros((16,), jnp.float32))
                        exh = jnp.exp(e + neg)
                        if heads == 8:
                            plsc.store_scatter(vbufC, [G + rows, colbase + h],
                                               exh)
                        else:
                            for hc in range(8):
                                plsc.store_scatter(vbufC,
                                                   [G + rows, colbase + hc],
                                                   exh)

                        def mv_body(t, _, h=h, exh=exh):
                            c = h * ch + t
                            cv = jnp.full((16,), c, jnp.int32)
                            xlv = plsc.load_gather(xb, [rows, cv])
                            plsc.store_scatter(vbufC, [rows, cv], xlv * exh)
                            return 0

                        lax.fori_loop(0, ch, mv_body, 0)
                    return 0

                lax.fori_loop(0, G // 16, sub_body, 0)
                pltpu.sync_copy(vbufC, accM.at[idxc], add=True)

                def zero_body(j, _):
                    rows = j * 16 + iota
                    dv = seld[pl.ds(g * G + j * 16, 16)]
                    colbase = ((dv - base) & 15) * 8
                    zv = jnp.zeros((16,), jnp.float32)
                    for h in range(8):
                        plsc.store_scatter(vbufC, [G + rows, colbase + h], zv)
                    return 0

                lax.fori_loop(0, G // 16, zero_body, 0)

            @pl.when(nsup > 0)
            def _():
                issue_gathers(0, 0)

            def pair_body(g2, _):
                g0 = g2 * 2

                @pl.when(g0 < nsup)
                def _():
                    @pl.when(g0 + 1 < nsup)
                    def _():
                        issue_gathers(g0 + 1, 1)

                    wait_gathers(0)
                    compute_and_scatter(g0, 0)

                g1 = g0 + 1

                @pl.when(g1 < nsup)
                def _():
                    @pl.when(g1 + 1 < nsup)
                    def _():
                        issue_gathers(g1 + 1, 0)

                    wait_gathers(1)
                    compute_and_scatter(g1, 1)

                return 0

            lax.fori_loop(0, (nsup + 1) // 2, pair_body, 0)
            plsc.subcore_barrier()

            pltpu.sync_copy(accM.at[pl.ds(wid * SHARE, SHARE)],
                            outM_hbm.at[pl.ds(base + wid * SHARE, SHARE)])

            @pl.when(wid == 0)
            def _():
                pltpu.sync_copy(accM.at[pl.ds(RNG, RPACK)],
                                outP_hbm.at[pl.ds(rid * RPACK, RPACK)])

            plsc.subcore_barrier()
            return 0

        lax.fori_loop(0, NRANGE // 2, range_body, 0)

    return sc_pass


_sc_pass_l1 = _make_sc_pass(8, 16)
_sc_pass_l2 = _make_sc_pass(1, 128)


def _gat_pass(sc_pass, xl, xr, src_p, dst_p, att, zm, b):
    outP, outM = sc_pass(xl, xr, src_p, dst_p, att.reshape(-1), zm)
    s = outP.reshape(NRANGE * RPACK * 16, 8)[:N]
    return _finalize(s, outM[:N], b)


def kernel(x_user, x_item, edge_index_ui, edge_index_iu,
           Wl1_ui, Wr1_ui, att1_ui, b1_ui,
           Wl1_iu, Wr1_iu, att1_iu, b1_iu,
           Wl2_ui, Wr2_ui, att2_ui, b2_ui,
           Wl2_iu, Wr2_iu, att2_iu, b2_iu,
           W_lin, b_lin):
    su, du = edge_index_ui[0], edge_index_ui[1]
    si, di = edge_index_iu[0], edge_index_iu[1]
    npad = EPAD - E
    zi = jnp.zeros((npad,), jnp.int32)
    sentinel = jnp.full((npad,), 1 << 30, jnp.int32)
    su_p = jnp.concatenate([su, zi])
    du_p = jnp.concatenate([du, sentinel])
    si_p = jnp.concatenate([si, zi])
    di_p = jnp.concatenate([di, sentinel])
    zm = jnp.zeros((SHARE, 128), jnp.float32)
    zb = jnp.zeros((128,), jnp.float32)

    xl_ui = _matmul_bias(x_user, Wl1_ui, zb)
    xr_ui = _matmul_bias(x_item, Wr1_ui, zb)
    xl_iu = _matmul_bias(x_item, Wl1_iu, zb)
    xr_iu = _matmul_bias(x_user, Wr1_iu, zb)
    h_item = _gat_pass(_sc_pass_l1, xl_ui, xr_ui, su_p, du_p, att1_ui, zm,
                       b1_ui)
    h_user = _gat_pass(_sc_pass_l1, xl_iu, xr_iu, si_p, di_p, att1_iu, zm,
                       b1_iu)

    xl2_ui = _matmul_bias(h_user, Wl2_ui, zb)
    xr2_ui = _matmul_bias(h_item, Wr2_ui, zb)
    xl2_iu = _matmul_bias(h_item, Wl2_iu, zb)
    xr2_iu = _matmul_bias(h_user, Wr2_iu, zb)
    o_item = _gat_pass(_sc_pass_l2, xl2_ui, xr2_ui, su_p, du_p, att2_ui, zm,
                       b2_ui)
    o_user = _gat_pass(_sc_pass_l2, xl2_iu, xr2_iu, si_p, di_p, att2_iu, zm,
                       b2_iu)

    out_user = _matmul_bias(o_user, W_lin, b_lin)
    out_item = _matmul_bias(o_item, W_lin, b_lin)
    return (out_user, out_item)


# speedup vs baseline: 1.0379x; 1.0379x over previous
import functools

import jax
import jax.numpy as jnp
from jax import lax
from jax.experimental import pallas as pl
from jax.experimental.pallas import tpu as pltpu
from jax.experimental.pallas import tpu_sc as plsc

N = 50000
E = 600000
D = 128
HID = 16
HEADS = 8
OUT = 128

NC = 2
NS = 16
NRANGE = 10
RNG = 5120
NOUT = NRANGE * RNG
RPACK = RNG // 16
ACCROWS = RNG + RPACK
EPW = 37504
EPAD = NS * EPW
NCHUNK = 8
CH = EPW // NCHUNK
SELCAP = 5120
G = 64
SHARE = RNG // NS


def _mm_kernel(x_ref, w_ref, b_ref, o_ref):
    o_ref[...] = jnp.dot(x_ref[...], w_ref[...],
                         preferred_element_type=jnp.float32) + b_ref[...]


def _matmul_bias(x, w, b):
    n, d = x.shape
    dout = w.shape[1]
    blk = 1000
    return pl.pallas_call(
        _mm_kernel,
        grid=(n // blk,),
        in_specs=[
            pl.BlockSpec((blk, d), lambda i: (i, 0)),
            pl.BlockSpec((d, dout), lambda i: (0, 0)),
            pl.BlockSpec((1, dout), lambda i: (0, 0)),
        ],
        out_specs=pl.BlockSpec((blk, dout), lambda i: (i, 0)),
        out_shape=jax.ShapeDtypeStruct((n, dout), jnp.float32),
    )(x, w, b.reshape(1, dout))


def _fin_kernel(s_ref, m_ref, b_ref, o_ref):
    s = s_ref[...]
    m = m_ref[...]
    parts = []
    for h in range(8):
        parts.append(m[:, h * 16:(h + 1) * 16] / (s[:, h:h + 1] + 1e-16))
    x = jnp.concatenate(parts, axis=1) + b_ref[...]
    o_ref[...] = jnp.where(x > 0, x, jnp.exp(jnp.minimum(x, 0.0)) - 1.0)


def _finalize(s, m, b):
    blk = 1000
    return pl.pallas_call(
        _fin_kernel,
        grid=(N // blk,),
        in_specs=[
            pl.BlockSpec((blk, 8), lambda i: (i, 0)),
            pl.BlockSpec((blk, 128), lambda i: (i, 0)),
            pl.BlockSpec((1, 128), lambda i: (0, 0)),
        ],
        out_specs=pl.BlockSpec((blk, 128), lambda i: (i, 0)),
        out_shape=jax.ShapeDtypeStruct((N, 128), jnp.float32),
    )(s, m, b.reshape(1, 128))


def _make_sc_pass(heads, ch):
    mesh = plsc.VectorSubcoreMesh(core_axis_name="c", subcore_axis_name="s")

    @functools.partial(
        pl.kernel,
        out_type=(jax.ShapeDtypeStruct((NRANGE * RPACK, 128), jnp.float32),
                  jax.ShapeDtypeStruct((NOUT, 128), jnp.float32)),
        mesh=mesh,
        compiler_params=pltpu.CompilerParams(needs_layout_passes=False),
        scratch_types=[
            pltpu.VMEM_SHARED((ACCROWS, 128), jnp.float32),
            pltpu.VMEM((CH,), jnp.int32),
            pltpu.VMEM((CH,), jnp.int32),
            pltpu.VMEM((SELCAP,), jnp.int32),
            pltpu.VMEM((SELCAP,), jnp.int32),
            pltpu.VMEM((G,), jnp.int32),
            pltpu.VMEM((G,), jnp.int32),
            pltpu.VMEM((G,), jnp.int32),
            pltpu.VMEM((G,), jnp.int32),
            pltpu.VMEM((G, 128), jnp.float32),
            pltpu.VMEM((G, 128), jnp.float32),
            pltpu.VMEM((G, 128), jnp.float32),
            pltpu.VMEM((G, 128), jnp.float32),
            pltpu.VMEM((2 * G, 128), jnp.float32),
            pltpu.VMEM((2 * G,), jnp.int32),
            pltpu.VMEM((128,), jnp.float32),
            pltpu.SemaphoreType.DMA,
            pltpu.SemaphoreType.DMA,
            pltpu.SemaphoreType.DMA,
            pltpu.SemaphoreType.DMA,
        ],
    )
    def sc_pass(xl_hbm, xr_hbm, src_hbm, dst_hbm, att_hbm, zm_hbm,
                outP_hbm, outM_hbm,
                accM, sbuf_s, sbuf_d, sels, seld,
                gidx_s0, gidx_d0, gidx_s1, gidx_d1,
                xlbuf0, xrbuf0, xlbuf1, xrbuf1,
                vbufC, idxc, attv,
                sem_l0, sem_r0, sem_l1, sem_r1):
        cid = lax.axis_index("c")
        wid = lax.axis_index("s")
        iota = lax.iota(jnp.int32, 16)

        pltpu.sync_copy(att_hbm, attv)
        pltpu.sync_copy(zm_hbm.at[pl.ds(0, G)], vbufC.at[pl.ds(G, G)])

        gbufs = ((gidx_s0, gidx_d0, xlbuf0, xrbuf0, sem_l0, sem_r0),
                 (gidx_s1, gidx_d1, xlbuf1, xrbuf1, sem_l1, sem_r1))

        def issue_gathers(g, p):
            gs, gd, xb, rb, sl, sr = gbufs[p]

            def idx_copy(t, _):
                gs[pl.ds(t * 16, 16)] = sels[pl.ds(g * G + t * 16, 16)]
                gd[pl.ds(t * 16, 16)] = seld[pl.ds(g * G + t * 16, 16)]
                return 0

            lax.fori_loop(0, G // 16, idx_copy, 0)
            pltpu.async_copy(xl_hbm.at[gs], xb, sl)
            pltpu.async_copy(xr_hbm.at[gd], rb, sr)

        def wait_gathers(p):
            gs, gd, xb, rb, sl, sr = gbufs[p]
            pltpu.make_async_copy(xl_hbm.at[gs], xb, sl).wait()
            pltpu.make_async_copy(xr_hbm.at[gd], rb, sr).wait()

        def range_body(r, _):
            base = (cid * (NRANGE // 2) + r) * RNG
            rid = cid * (NRANGE // 2) + r
            pltpu.sync_copy(zm_hbm.at[pl.ds(0, SHARE)],
                            accM.at[pl.ds(wid * SHARE, SHARE)])

            @pl.when(wid == 0)
            def _():
                pltpu.sync_copy(zm_hbm.at[pl.ds(0, RPACK)],
                                accM.at[pl.ds(RNG, RPACK)])

            plsc.subcore_barrier()

            def chunk_body(k, cursor):
                off = wid * EPW + k * CH
                pltpu.sync_copy(src_hbm.at[pl.ds(off, CH)], sbuf_s)
                pltpu.sync_copy(dst_hbm.at[pl.ds(off, CH)], sbuf_d)

                def vec_body(i, cur):
                    dv = sbuf_d[pl.ds(i * 16, 16)]
                    sv = sbuf_s[pl.ds(i * 16, 16)]
                    m = (dv >= base) & (dv < base + RNG)
                    mi = m.astype(jnp.int32)
                    cum = plsc.cumsum(mi)
                    pos = cur + cum - 1
                    pos = jnp.where(m, pos, SELCAP - 16 + iota)
                    plsc.store_scatter(seld, [pos], dv, mask=m)
                    plsc.store_scatter(sels, [pos], sv, mask=m)
                    return jnp.minimum(cur + cum[15], SELCAP - G - 16)

                return lax.fori_loop(0, CH // 16, vec_body, cursor)

            cnt = lax.fori_loop(0, NCHUNK, chunk_body, 0)

            padd = jnp.full((16,), base, jnp.int32)
            pads = jnp.zeros((16,), jnp.int32)

            def pad_body(k, _):
                seld[pl.ds(cnt + k * 16, 16)] = padd
                sels[pl.ds(cnt + k * 16, 16)] = pads
                return 0

            lax.fori_loop(0, G // 16, pad_body, 0)

            nsup = (cnt + G - 1) // G

            def compute_and_scatter(g, p):
                _, _, xb, rb, _, _ = gbufs[p]

                def sub_body(j, _):
                    rows = j * 16 + iota
                    dv = seld[pl.ds(g * G + j * 16, 16)]
                    dstl = dv - base
                    idxc[pl.ds(j * 16, 16)] = dstl
                    idxc[pl.ds(G + j * 16, 16)] = RNG + (dstl >> 4)
                    colbase = (dstl & 15) * 8
                    valid = (g * G + j * 16 + iota) < cnt
                    neg = jnp.where(valid, 0.0, -1e30)
                    for h in range(heads):
                        def ch_body(t, eacc, h=h):
                            c = h * ch + t
                            cv = jnp.full((16,), c, jnp.int32)
                            av = plsc.load_gather(xb, [rows, cv])
                            bv = plsc.load_gather(rb, [rows, cv])
                            atv = plsc.load_gather(attv, [cv])
                            xe = av + bv
                            lr = jnp.maximum(xe, xe * 0.2)
                            return eacc + lr * atv

                        e = lax.fori_loop(0, ch, ch_body,
                                          jnp.zeros((16,), jnp.float32))
                        exh = jnp.exp(e + neg)
                        if heads == 8:
                            plsc.store_scatter(vbufC, [G + rows, colbase + h],
                                               exh)
                        else:
                            for hc in range(8):
                                plsc.store_scatter(vbufC,
                                                   [G + rows, colbase + hc],
                                                   exh)

                        def mv_body(t, _, h=h, exh=exh):
                            c = h * ch + t
                            cv = jnp.full((16,), c, jnp.int32)
                            xlv = plsc.load_gather(xb, [rows, cv])
                            plsc.store_scatter(vbufC, [rows, cv], xlv * exh)
                            return 0

                        lax.fori_loop(0, ch, mv_body, 0)
                    return 0

                lax.fori_loop(0, G // 16, sub_body, 0)
                pltpu.sync_copy(vbufC, accM.at[idxc], add=True)

                def zero_body(j, _):
                    rows = j * 16 + iota
                    dv = seld[pl.ds(g * G + j * 16, 16)]
                    colbase = ((dv - base) & 15) * 8
                    zv = jnp.zeros((16,), jnp.float32)
                    for h in range(8):
                        plsc.store_scatter(vbufC, [G + rows, colbase + h], zv)
                    return 0

                lax.fori_loop(0, G // 16, zero_body, 0)

            @pl.when(nsup > 0)
            def _():
                issue_gathers(0, 0)

            def pair_body(g2, _):
                g0 = g2 * 2

                @pl.when(g0 < nsup)
                def _():
                    @pl.when(g0 + 1 < nsup)
                    def _():
                        issue_gathers(g0 + 1, 1)

                    wait_gathers(0)
                    compute_and_scatter(g0, 0)

                g1 = g0 + 1

                @pl.when(g1 < nsup)
                def _():
                    @pl.when(g1 + 1 < nsup)
                    def _():
                        issue_gathers(g1 + 1, 0)

                    wait_gathers(1)
                    compute_and_scatter(g1, 1)

                return 0

            lax.fori_loop(0, (nsup + 1) // 2, pair_body, 0)
            plsc.subcore_barrier()

            pltpu.sync_copy(accM.at[pl.ds(wid * SHARE, SHARE)],
                            outM_hbm.at[pl.ds(base + wid * SHARE, SHARE)])

            @pl.when(wid == 0)
            def _():
                pltpu.sync_copy(accM.at[pl.ds(RNG, RPACK)],
                                outP_hbm.at[pl.ds(rid * RPACK, RPACK)])

            plsc.subcore_barrier()
            return 0

        lax.fori_loop(0, NRANGE // 2, range_body, 0)

    return sc_pass


_sc_pass_l1 = _make_sc_pass(8, 16)
_sc_pass_l2 = _make_sc_pass(1, 128)


def _gat_pass(sc_pass, xl, xr, src_p, dst_p, att, zm, b):
    outP, outM = sc_pass(xl, xr, src_p, dst_p, att.reshape(-1), zm)
    s = outP.reshape(NRANGE * RPACK * 16, 8)[:N]
    return _finalize(s, outM[:N], b)


def kernel(x_user, x_item, edge_index_ui, edge_index_iu,
           Wl1_ui, Wr1_ui, att1_ui, b1_ui,
           Wl1_iu, Wr1_iu, att1_iu, b1_iu,
           Wl2_ui, Wr2_ui, att2_ui, b2_ui,
           Wl2_iu, Wr2_iu, att2_iu, b2_iu,
           W_lin, b_lin):
    su, du = edge_index_ui[0], edge_index_ui[1]
    si, di = edge_index_iu[0], edge_index_iu[1]
    npad = EPAD - E
    zi = jnp.zeros((npad,), jnp.int32)
    sentinel = jnp.full((npad,), 1 << 30, jnp.int32)
    su_p = jnp.concatenate([su, zi])
    du_p = jnp.concatenate([du, sentinel])
    si_p = jnp.concatenate([si, zi])
    di_p = jnp.concatenate([di, sentinel])
    zm = jnp.zeros((SHARE, 128), jnp.float32)
    zb = jnp.zeros((128,), jnp.float32)

    xl_ui = _matmul_bias(x_user, Wl1_ui, zb)
    xr_ui = _matmul_bias(x_item, Wr1_ui, zb)
    xl_iu = _matmul_bias(x_item, Wl1_iu, zb)
    xr_iu = _matmul_bias(x_user, Wr1_iu, zb)
    h_item = _gat_pass(_sc_pass_l1, xl_ui, xr_ui, su_p, du_p, att1_ui, zm,
                       b1_ui)
    h_user = _gat_pass(_sc_pass_l1, xl_iu, xr_iu, si_p, di_p, att1_iu, zm,
                       b1_iu)

    xl2_ui = _matmul_bias(h_user, Wl2_ui, zb)
    xr2_ui = _matmul_bias(h_item, Wr2_ui, zb)
    xl2_iu = _matmul_bias(h_item, Wl2_iu, zb)
    xr2_iu = _matmul_bias(h_user, Wr2_iu, zb)
    o_item = _gat_pass(_sc_pass_l2, xl2_ui, xr2_ui, su_p, du_p, att2_ui, zm,
                       b2_ui)
    o_user = _gat_pass(_sc_pass_l2, xl2_iu, xr2_iu, si_p, di_p, att2_iu, zm,
                       b2_iu)

    out_user = _matmul_bias(o_user, W_lin, b_lin)
    out_item = _matmul_bias(o_item, W_lin, b_lin)
    return (out_user, out_item)
